# baseline (speedup 1.0000x reference)
import functools

import jax
import jax.numpy as jnp
from jax import lax
from jax.experimental import pallas as pl
from jax.experimental.pallas import tpu as pltpu
from jax.experimental.pallas import tpu_sc as plsc

B = 128


def _node_enc_kernel(x_ref, wn_ref, bn_ref, g_ref, be_ref, h0_ref, h_ref):
    h0 = jnp.dot(x_ref[...], wn_ref[...], preferred_element_type=jnp.float32) + bn_ref[...]
    h0_ref[...] = h0
    mean = jnp.mean(h0, axis=0, keepdims=True)
    var = jnp.mean((h0 - mean) ** 2, axis=0, keepdims=True)
    h_ref[...] = jnp.maximum(
        (h0 - mean) / jnp.sqrt(var + 1e-5) * g_ref[...] + be_ref[...], 0.0)


def _edge_enc_kernel(eattr_ref, we_ref, be_ref, o_ref):
    o_ref[...] = (
        jnp.dot(eattr_ref[...], we_ref[...], preferred_element_type=jnp.float32)
        + be_ref[...]
    )


def _mlp_kernel(acc_ref, h_ref, h0_ref, w1_ref, b1_ref, lg_ref, lb_ref, w2_ref,
                b2_ref, xf_ref):
    agg = acc_ref[0] / (acc_ref[1] + 1e-16)
    out = agg + h_ref[...]
    y = jnp.dot(out, w1_ref[...], preferred_element_type=jnp.float32) + b1_ref[...]
    mean = jnp.mean(y, axis=-1, keepdims=True)
    var = jnp.mean((y - mean) ** 2, axis=-1, keepdims=True)
    y = jnp.maximum((y - mean) / jnp.sqrt(var + 1e-5) * lg_ref[...] + lb_ref[...], 0.0)
    y = jnp.dot(y, w2_ref[...], preferred_element_type=jnp.float32) + b2_ref[...]
    xf_ref[...] = h0_ref[...] + y


def _make_agg_call(n_nodes, n_edges):
    bb = 80
    nch = n_edges // bb
    trips = -(-nch // 16)
    nrch = -(-n_nodes // bb)
    assert n_nodes % bb == 0 and n_edges % bb == 0
    rtrips = -(-nrch // 16)
    mesh = plsc.VectorSubcoreMesh(core_axis_name="c", subcore_axis_name="s")

    @functools.partial(
        pl.kernel,
        mesh=mesh,
        out_type=jax.ShapeDtypeStruct((2, n_nodes, 128), jnp.float32),
        scratch_types=[
            pltpu.VMEM((2, bb), jnp.int32),
            pltpu.VMEM((2, bb), jnp.int32),
            pltpu.VMEM((2, bb, 128), jnp.float32),
            pltpu.VMEM((2, bb, 128), jnp.float32),
            pltpu.VMEM((16,), jnp.float32),
            pltpu.VMEM_SHARED((n_nodes, 128), jnp.float32),
            pltpu.SemaphoreType.DMA,
            pltpu.SemaphoreType.DMA,
            pltpu.SemaphoreType.DMA,
        ],
        compiler_params=pltpu.CompilerParams(needs_layout_passes=False),
    )
    def agg_call(src_hbm, dst_hbm, h_hbm, ea_hbm, t_hbm, out_hbm,
                 sidx, didx, hbuf, eabuf, tbuf, acc, semi, semb0, semb1):
        c = lax.axis_index("c")
        s = lax.axis_index("s")
        semb = (semb0, semb1)

        zv = jnp.zeros((16,), jnp.float32)

        @plsc.parallel_loop(0, bb, 1, unroll=4)
        def zrow(i):
            for j in range(8):
                hbuf[0, i, pl.ds(j * 16, 16)] = zv

        def zchunk(k, carry):
            rid = k * 16 + s

            @pl.when(rid < nrch)
            def _():
                pltpu.sync_copy(hbuf.at[0], acc.at[pl.ds(rid * bb, bb)])

            return carry

        lax.fori_loop(0, rtrips, zchunk, 0)
        plsc.subcore_barrier()

        pltpu.sync_copy(t_hbm, tbuf)
        tv = tbuf[...]
        nchm1 = nch - 1

        def issue_idx(k, b):
            base = jnp.minimum(k * 16 + s, nchm1) * bb
            pltpu.async_copy(src_hbm.at[pl.ds(base, bb)], sidx.at[b], semi)
            pltpu.async_copy(dst_hbm.at[pl.ds(base, bb)], didx.at[b], semi)

        def wait_idx(b):
            pltpu.make_async_copy(src_hbm.at[pl.ds(0, bb)], sidx.at[b], semi).wait()
            pltpu.make_async_copy(dst_hbm.at[pl.ds(0, bb)], didx.at[b], semi).wait()

        def issue_data(k, b):
            base = jnp.minimum(k * 16 + s, nchm1) * bb
            pltpu.async_copy(h_hbm.at[sidx.at[b]], hbuf.at[b], semb[b])
            pltpu.async_copy(ea_hbm.at[pl.ds(base, bb), :], eabuf.at[b], semb[b])

        def wait_data(b):
            pltpu.make_async_copy(h_hbm.at[sidx.at[b]], hbuf.at[b], semb[b]).wait()
            pltpu.make_async_copy(ea_hbm.at[pl.ds(0, bb), :], eabuf.at[b], semb[b]).wait()

        issue_idx(0, 0)
        wait_idx(0)
        issue_data(0, 0)
        issue_idx(1, 1)
        trips2 = trips + (trips % 2)

        def pair(k2, carry):
            for b in range(2):
                k = k2 * 2 + b
                wait_idx(1 - b)
                issue_data(k + 1, 1 - b)
                wait_data(b)

                @pl.when(c == 0)
                def _():
                    @plsc.parallel_loop(0, bb, 1, unroll=4)
                    def row(i):
                        for j in range(8):
                            hv = hbuf[b, i, pl.ds(j * 16, 16)]
                            ev = eabuf[b, i, pl.ds(j * 16, 16)]
                            m = jnp.maximum(hv + ev, 0.0) + 1e-7
                            hbuf[b, i, pl.ds(j * 16, 16)] = jnp.exp(tv * m) * m

                @pl.when(c == 1)
                def _():
                    @plsc.parallel_loop(0, bb, 1, unroll=4)
                    def row(i):
                        for j in range(8):
                            hv = hbuf[b, i, pl.ds(j * 16, 16)]
                            ev = eabuf[b, i, pl.ds(j * 16, 16)]
                            m = jnp.maximum(hv + ev, 0.0) + 1e-7
                            hbuf[b, i, pl.ds(j * 16, 16)] = jnp.exp(tv * m)

                cid = k * 16 + s

                @pl.when(cid < nch)
                def _():
                    pltpu.sync_copy(hbuf.at[b], acc.at[didx.at[b]], add=True)

                issue_idx(k + 2, b)
            return carry

        lax.fori_loop(0, trips2 // 2, pair, 0)
        wait_data(0)
        wait_idx(1)
        plsc.subcore_barrier()

        def wchunk(k, carry):
            rid = k * 16 + s

            @pl.when(rid < nrch)
            def _():
                pltpu.sync_copy(acc.at[pl.ds(rid * bb, bb)],
                                out_hbm.at[c, pl.ds(rid * bb, bb), :])

            return carry

        lax.fori_loop(0, rtrips, wchunk, 0)

    return agg_call


def _make_dot_call(n_edges):
    nch = n_edges // B
    trips = -(-nch // 32)
    mesh = plsc.VectorSubcoreMesh(core_axis_name="c", subcore_axis_name="s")

    @functools.partial(
        pl.kernel,
        mesh=mesh,
        out_type=jax.ShapeDtypeStruct((n_edges,), jnp.float32),
        scratch_types=[
            pltpu.VMEM((2, B), jnp.int32),
            pltpu.VMEM((2, B), jnp.int32),
            pltpu.VMEM((2, B, 128), jnp.float32),
            pltpu.VMEM((2, B, 128), jnp.float32),
            pltpu.VMEM((B,), jnp.float32),
            pltpu.VMEM((B * 16,), jnp.float32),
            pltpu.SemaphoreType.DMA,
            pltpu.SemaphoreType.DMA,
            pltpu.SemaphoreType.DMA,
        ],
        compiler_params=pltpu.CompilerParams(needs_layout_passes=False),
    )
    def dot_call(src_hbm, dst_hbm, xf_hbm, out_hbm, sidx, didx, abuf, bbuf,
                 obuf, pbuf, semi, semb0, semb1):
        c = lax.axis_index("c")
        s = lax.axis_index("s")
        w = s * 2 + c
        semb = (semb0, semb1)
        nchm1 = nch - 1
        lane16 = lax.broadcasted_iota(jnp.int32, (16,), 0) * 16

        def issue_idx(k, b):
            base = jnp.minimum(k * 32 + w, nchm1) * B
            pltpu.async_copy(src_hbm.at[pl.ds(base, B)], sidx.at[b], semi)
            pltpu.async_copy(dst_hbm.at[pl.ds(base, B)], didx.at[b], semi)

        def wait_idx(b):
            pltpu.make_async_copy(src_hbm.at[pl.ds(0, B)], sidx.at[b], semi).wait()
            pltpu.make_async_copy(dst_hbm.at[pl.ds(0, B)], didx.at[b], semi).wait()

        def issue_data(k, b):
            pltpu.async_copy(xf_hbm.at[sidx.at[b]], abuf.at[b], semb[b])
            pltpu.async_copy(xf_hbm.at[didx.at[b]], bbuf.at[b], semb[b])

        def wait_data(b):
            pltpu.make_async_copy(xf_hbm.at[sidx.at[b]], abuf.at[b], semb[b]).wait()
            pltpu.make_async_copy(xf_hbm.at[didx.at[b]], bbuf.at[b], semb[b]).wait()

        issue_idx(0, 0)
        wait_idx(0)
        issue_data(0, 0)
        issue_idx(1, 1)
        trips2 = trips + (trips % 2)

        def pair(k2, carry):
            for b in range(2):
                k = k2 * 2 + b
                wait_idx(1 - b)
                issue_data(k + 1, 1 - b)
                wait_data(b)

                @plsc.parallel_loop(0, B // 16, 1, unroll=2)
                def grp(g):
                    pb = g * 256
                    for i2 in range(16):
                        i = g * 16 + i2
                        accv = abuf[b, i, pl.ds(0, 16)] * bbuf[b, i, pl.ds(0, 16)]
                        for j in range(1, 8):
                            accv = accv + abuf[b, i, pl.ds(j * 16, 16)] * bbuf[b, i, pl.ds(j * 16, 16)]
                        pbuf[pl.ds(pb + i2 * 16, 16)] = accv
                    tot = plsc.load_gather(pbuf, [pb + lane16])
                    for j in range(1, 16):
                        tot = tot + plsc.load_gather(pbuf, [pb + lane16 + j])
                    obuf[pl.ds(g * 16, 16)] = tot

                cid = k * 32 + w

                @pl.when(cid < nch)
                def _():
                    pltpu.sync_copy(obuf, out_hbm.at[pl.ds(cid * B, B)])

                issue_idx(k + 2, b)
            return carry

        lax.fori_loop(0, trips2 // 2, pair, 0)
        wait_data(0)
        wait_idx(1)

    return dot_call


def kernel(x, edge_index, edge_attr, W_node, b_node, W_edge, b_edge,
           bn_gamma, bn_beta, t, W1, b1, ln_gamma, ln_beta, W2, b2):
    n, d = x.shape[0], W_node.shape[1]
    e = edge_index.shape[1]
    src = edge_index[0]
    dst = edge_index[1]

    h0, h = pl.pallas_call(
        _node_enc_kernel,
        out_shape=[
            jax.ShapeDtypeStruct((n, d), jnp.float32),
            jax.ShapeDtypeStruct((n, d), jnp.float32),
        ],
    )(x, W_node, b_node.reshape(1, d), bn_gamma.reshape(1, d),
      bn_beta.reshape(1, d))

    eb = 8000
    ea = pl.pallas_call(
        _edge_enc_kernel,
        grid=(e // eb,),
        in_specs=[
            pl.BlockSpec((eb, 16), lambda i: (i, 0)),
            pl.BlockSpec((16, d), lambda i: (0, 0)),
            pl.BlockSpec((1, d), lambda i: (0, 0)),
        ],
        out_specs=pl.BlockSpec((eb, d), lambda i: (i, 0)),
        out_shape=jax.ShapeDtypeStruct((e, d), jnp.float32),
    )(edge_attr, W_edge, b_edge.reshape(1, d))

    t16 = jnp.full((16,), t, dtype=jnp.float32)
    acc = _make_agg_call(n, e)(src, dst, h, ea, t16)

    xf = pl.pallas_call(
        _mlp_kernel,
        out_shape=jax.ShapeDtypeStruct((n, d), jnp.float32),
    )(acc, h, h0, W1, b1.reshape(1, 2 * d), ln_gamma.reshape(1, 2 * d),
      ln_beta.reshape(1, 2 * d), W2, b2.reshape(1, d))

    return _make_dot_call(e)(src, dst, xf)


# speedup vs baseline: 1.1560x; 1.1560x over previous
import functools

import jax
import jax.numpy as jnp
from jax import lax
from jax.experimental import pallas as pl
from jax.experimental.pallas import tpu as pltpu
from jax.experimental.pallas import tpu_sc as plsc

B = 128


def _node_enc_kernel(x_ref, wn_ref, bn_ref, g_ref, be_ref, h0_ref, h_ref):
    h0 = jnp.dot(x_ref[...], wn_ref[...], preferred_element_type=jnp.float32) + bn_ref[...]
    h0_ref[...] = h0
    mean = jnp.mean(h0, axis=0, keepdims=True)
    var = jnp.mean((h0 - mean) ** 2, axis=0, keepdims=True)
    h_ref[...] = jnp.maximum(
        (h0 - mean) / jnp.sqrt(var + 1e-5) * g_ref[...] + be_ref[...], 0.0)


def _edge_enc_kernel(eattr_ref, we_ref, be_ref, o_ref):
    a = eattr_ref[...]
    o_ref[0] = (
        jnp.dot(a, we_ref[0], preferred_element_type=jnp.float32) + be_ref[0]
    )
    o_ref[1] = (
        jnp.dot(a, we_ref[1], preferred_element_type=jnp.float32) + be_ref[1]
    )


def _mlp_kernel(acc_ref, h_ref, h0_ref, w1_ref, b1_ref, lg_ref, lb_ref, w2_ref,
                b2_ref, xf_ref):
    agg = jnp.concatenate(
        [acc_ref[0, :, 0:64] / (acc_ref[0, :, 64:128] + 1e-16),
         acc_ref[1, :, 0:64] / (acc_ref[1, :, 64:128] + 1e-16)], axis=1)
    out = agg + h_ref[...]
    y = jnp.dot(out, w1_ref[...], preferred_element_type=jnp.float32) + b1_ref[...]
    mean = jnp.mean(y, axis=-1, keepdims=True)
    var = jnp.mean((y - mean) ** 2, axis=-1, keepdims=True)
    y = jnp.maximum((y - mean) / jnp.sqrt(var + 1e-5) * lg_ref[...] + lb_ref[...], 0.0)
    y = jnp.dot(y, w2_ref[...], preferred_element_type=jnp.float32) + b2_ref[...]
    xf_ref[...] = h0_ref[...] + y


def _make_agg_call(n_nodes, n_edges):
    bb = 64
    nch = n_edges // bb
    trips = -(-nch // 16)
    nrfull, nrrem = divmod(n_nodes, bb)
    nrch = nrfull + (1 if nrrem else 0)
    assert n_edges % bb == 0 and nrrem % 8 == 0
    rtrips = -(-nrch // 16)
    mesh = plsc.VectorSubcoreMesh(core_axis_name="c", subcore_axis_name="s")

    @functools.partial(
        pl.kernel,
        mesh=mesh,
        out_type=jax.ShapeDtypeStruct((2, n_nodes, 128), jnp.float32),
        scratch_types=[
            pltpu.VMEM((2, bb), jnp.int32),
            pltpu.VMEM((2, bb), jnp.int32),
            pltpu.VMEM((2, bb, 128), jnp.float32),
            pltpu.VMEM((2, bb // 2, 128), jnp.float32),
            pltpu.VMEM((2, bb, 128), jnp.float32),
            pltpu.VMEM((16,), jnp.float32),
            pltpu.VMEM_SHARED((n_nodes, 128), jnp.float32),
            pltpu.SemaphoreType.DMA,
            pltpu.SemaphoreType.DMA,
            pltpu.SemaphoreType.DMA,
        ],
        compiler_params=pltpu.CompilerParams(needs_layout_passes=False),
    )
    def agg_call(src_hbm, dst_hbm, h_hbm, ea_hbm, t_hbm, out_hbm,
                 sidx, didx, hbuf, eabuf, upd, tbuf, acc, semi, semb0, semb1):
        c = lax.axis_index("c")
        s = lax.axis_index("s")
        semb = (semb0, semb1)

        zv = jnp.zeros((16,), jnp.float32)

        @plsc.parallel_loop(0, bb, 1, unroll=4)
        def zrow(i):
            for j in range(8):
                upd[0, i, pl.ds(j * 16, 16)] = zv

        def zchunk(k, carry):
            rid = k * 16 + s

            @pl.when(rid < nrfull)
            def _():
                pltpu.sync_copy(upd.at[0], acc.at[pl.ds(rid * bb, bb)])

            if nrrem:
                @pl.when(rid == nrfull)
                def _():
                    pltpu.sync_copy(upd.at[0, pl.ds(0, nrrem)],
                                    acc.at[pl.ds(nrfull * bb, nrrem)])
            return carry

        lax.fori_loop(0, rtrips, zchunk, 0)
        plsc.subcore_barrier()

        pltpu.sync_copy(t_hbm, tbuf)
        tv = tbuf[...]
        nchm1 = nch - 1

        def issue_idx(k, b):
            base = jnp.minimum(k * 16 + s, nchm1) * bb
            pltpu.async_copy(src_hbm.at[pl.ds(base, bb)], sidx.at[b], semi)
            pltpu.async_copy(dst_hbm.at[pl.ds(base, bb)], didx.at[b], semi)

        def wait_idx(b):
            pltpu.make_async_copy(src_hbm.at[pl.ds(0, bb)], sidx.at[b], semi).wait()
            pltpu.make_async_copy(dst_hbm.at[pl.ds(0, bb)], didx.at[b], semi).wait()

        def issue_data(k, b):
            pbase = jnp.minimum(k * 16 + s, nchm1) * (bb // 2)
            pltpu.async_copy(h_hbm.at[sidx.at[b]], hbuf.at[b], semb[b])
            pltpu.async_copy(ea_hbm.at[c, pl.ds(pbase, bb // 2), :], eabuf.at[b],
                             semb[b])

        def wait_data(b):
            pltpu.make_async_copy(h_hbm.at[sidx.at[b]], hbuf.at[b], semb[b]).wait()
            pltpu.make_async_copy(ea_hbm.at[c, pl.ds(0, bb // 2), :], eabuf.at[b],
                                  semb[b]).wait()

        hoff = c * 64

        issue_idx(0, 0)
        wait_idx(0)
        issue_data(0, 0)
        issue_idx(1, 1)
        trips2 = trips + (trips % 2)

        def pair(k2, carry):
            for b in range(2):
                k = k2 * 2 + b
                wait_idx(1 - b)
                issue_data(k + 1, 1 - b)
                wait_data(b)

                @plsc.parallel_loop(0, bb // 2, 1, unroll=4)
                def pairrow(p):
                    for half in range(2):
                        i = 2 * p + half
                        for j in range(4):
                            hv = hbuf[b, i, pl.ds(hoff + j * 16, 16)]
                            ev = eabuf[b, p, pl.ds(64 * half + j * 16, 16)]
                            m = jnp.maximum(hv + ev, 0.0) + 1e-7
                            x = jnp.exp(tv * m)
                            upd[b, i, pl.ds(j * 16, 16)] = x * m
                            upd[b, i, pl.ds(64 + j * 16, 16)] = x

                cid = k * 16 + s

                @pl.when(cid < nch)
                def _():
                    pltpu.sync_copy(upd.at[b], acc.at[didx.at[b]], add=True)

                issue_idx(k + 2, b)
            return carry

        lax.fori_loop(0, trips2 // 2, pair, 0)
        wait_data(0)
        wait_idx(1)
        plsc.subcore_barrier()

        def wchunk(k, carry):
            rid = k * 16 + s

            @pl.when(rid < nrfull)
            def _():
                pltpu.sync_copy(acc.at[pl.ds(rid * bb, bb)],
                                out_hbm.at[c, pl.ds(rid * bb, bb), :])

            if nrrem:
                @pl.when(rid == nrfull)
                def _():
                    pltpu.sync_copy(acc.at[pl.ds(nrfull * bb, nrrem)],
                                    out_hbm.at[c, pl.ds(nrfull * bb, nrrem), :])
            return carry

        lax.fori_loop(0, rtrips, wchunk, 0)

    return agg_call


def _make_dot_call(n_edges):
    nch = n_edges // B
    trips = -(-nch // 32)
    mesh = plsc.VectorSubcoreMesh(core_axis_name="c", subcore_axis_name="s")

    @functools.partial(
        pl.kernel,
        mesh=mesh,
        out_type=jax.ShapeDtypeStruct((n_edges,), jnp.float32),
        scratch_types=[
            pltpu.VMEM((2, B), jnp.int32),
            pltpu.VMEM((2, B), jnp.int32),
            pltpu.VMEM((2, B, 128), jnp.float32),
            pltpu.VMEM((2, B, 128), jnp.float32),
            pltpu.VMEM((B,), jnp.float32),
            pltpu.VMEM((B * 16,), jnp.float32),
            pltpu.SemaphoreType.DMA,
            pltpu.SemaphoreType.DMA,
            pltpu.SemaphoreType.DMA,
        ],
        compiler_params=pltpu.CompilerParams(needs_layout_passes=False),
    )
    def dot_call(src_hbm, dst_hbm, xf_hbm, out_hbm, sidx, didx, abuf, bbuf,
                 obuf, pbuf, semi, semb0, semb1):
        c = lax.axis_index("c")
        s = lax.axis_index("s")
        w = s * 2 + c
        semb = (semb0, semb1)
        nchm1 = nch - 1
        lane16 = lax.broadcasted_iota(jnp.int32, (16,), 0) * 16

        def issue_idx(k, b):
            base = jnp.minimum(k * 32 + w, nchm1) * B
            pltpu.async_copy(src_hbm.at[pl.ds(base, B)], sidx.at[b], semi)
            pltpu.async_copy(dst_hbm.at[pl.ds(base, B)], didx.at[b], semi)

        def wait_idx(b):
            pltpu.make_async_copy(src_hbm.at[pl.ds(0, B)], sidx.at[b], semi).wait()
            pltpu.make_async_copy(dst_hbm.at[pl.ds(0, B)], didx.at[b], semi).wait()

        def issue_data(k, b):
            pltpu.async_copy(xf_hbm.at[sidx.at[b]], abuf.at[b], semb[b])
            pltpu.async_copy(xf_hbm.at[didx.at[b]], bbuf.at[b], semb[b])

        def wait_data(b):
            pltpu.make_async_copy(xf_hbm.at[sidx.at[b]], abuf.at[b], semb[b]).wait()
            pltpu.make_async_copy(xf_hbm.at[didx.at[b]], bbuf.at[b], semb[b]).wait()

        issue_idx(0, 0)
        wait_idx(0)
        issue_data(0, 0)
        issue_idx(1, 1)
        trips2 = trips + (trips % 2)

        def pair(k2, carry):
            for b in range(2):
                k = k2 * 2 + b
                wait_idx(1 - b)
                issue_data(k + 1, 1 - b)
                wait_data(b)

                @plsc.parallel_loop(0, B // 16, 1, unroll=2)
                def grp(g):
                    pb = g * 256
                    for i2 in range(16):
                        i = g * 16 + i2
                        accv = abuf[b, i, pl.ds(0, 16)] * bbuf[b, i, pl.ds(0, 16)]
                        for j in range(1, 8):
                            accv = accv + abuf[b, i, pl.ds(j * 16, 16)] * bbuf[b, i, pl.ds(j * 16, 16)]
                        pbuf[pl.ds(pb + i2 * 16, 16)] = accv
                    tot = plsc.load_gather(pbuf, [pb + lane16])
                    for j in range(1, 16):
                        tot = tot + plsc.load_gather(pbuf, [pb + lane16 + j])
                    obuf[pl.ds(g * 16, 16)] = tot

                cid = k * 32 + w

                @pl.when(cid < nch)
                def _():
                    pltpu.sync_copy(obuf, out_hbm.at[pl.ds(cid * B, B)])

                issue_idx(k + 2, b)
            return carry

        lax.fori_loop(0, trips2 // 2, pair, 0)
        wait_data(0)
        wait_idx(1)

    return dot_call


def kernel(x, edge_index, edge_attr, W_node, b_node, W_edge, b_edge,
           bn_gamma, bn_beta, t, W1, b1, ln_gamma, ln_beta, W2, b2):
    n, d = x.shape[0], W_node.shape[1]
    e = edge_index.shape[1]
    src = edge_index[0]
    dst = edge_index[1]

    h0, h = pl.pallas_call(
        _node_enc_kernel,
        out_shape=[
            jax.ShapeDtypeStruct((n, d), jnp.float32),
            jax.ShapeDtypeStruct((n, d), jnp.float32),
        ],
    )(x, W_node, b_node.reshape(1, d), bn_gamma.reshape(1, d),
      bn_beta.reshape(1, d))

    wz = jnp.zeros((16, 64), jnp.float32)
    W_pair = jnp.stack([
        jnp.block([[W_edge[:, 0:64], wz], [wz, W_edge[:, 0:64]]]),
        jnp.block([[W_edge[:, 64:128], wz], [wz, W_edge[:, 64:128]]]),
    ])
    b_pair = jnp.stack([
        jnp.concatenate([b_edge[0:64], b_edge[0:64]]).reshape(1, d),
        jnp.concatenate([b_edge[64:128], b_edge[64:128]]).reshape(1, d),
    ])

    eb = 4000
    ea = pl.pallas_call(
        _edge_enc_kernel,
        grid=(e // 2 // eb,),
        in_specs=[
            pl.BlockSpec((eb, 32), lambda i: (i, 0)),
            pl.BlockSpec((2, 32, d), lambda i: (0, 0, 0)),
            pl.BlockSpec((2, 1, d), lambda i: (0, 0, 0)),
        ],
        out_specs=pl.BlockSpec((2, eb, d), lambda i: (0, i, 0)),
        out_shape=jax.ShapeDtypeStruct((2, e // 2, d), jnp.float32),
    )(edge_attr.reshape(e // 2, 32), W_pair, b_pair)

    t16 = jnp.full((16,), t, dtype=jnp.float32)
    acc = _make_agg_call(n, e)(src, dst, h, ea, t16)

    xf = pl.pallas_call(
        _mlp_kernel,
        out_shape=jax.ShapeDtypeStruct((n, d), jnp.float32),
    )(acc, h, h0, W1, b1.reshape(1, 2 * d), ln_gamma.reshape(1, 2 * d),
      ln_beta.reshape(1, 2 * d), W2, b2.reshape(1, d))

    return _make_dot_call(e)(src, dst, xf)
